# Initial kernel scaffold; baseline (speedup 1.0000x reference)
import jax
import jax.numpy as jnp
from jax.experimental import pallas as pl


def kernel(x, emb_tables, W1, b1, W2, b2, W3, b3, g1, be1, g2, be2, g3, be3):
    raise NotImplementedError("write your pallas kernel here")


# speedup vs baseline: 1.1402x; 1.1402x over previous
import functools

import jax
import jax.numpy as jnp
from jax import lax
from jax.experimental import pallas as pl
from jax.experimental.pallas import tpu as pltpu
from jax.experimental.pallas import tpu_sc as plsc

B = 4096
NCAT = 26
NCONT = 13
VOCAB = 100000
ED = 32
L1 = 512
L2 = 256
NC = 2
N_EMB = NCAT * ED
EPS = 1e-5

SC_CORES = 2
SC_SUBCORES = 16
NW = SC_CORES * SC_SUBCORES
TOTAL_ROWS = B * NCAT
ROWS_PER_W = TOTAL_ROWS // NW
IDX_MINOR = 128
CHUNKS_PER_W = ROWS_PER_W // IDX_MINOR


def _gather_body(tables_hbm, idx_hbm, out_hbm, idx_v, rows_v, sem):
    wid = lax.axis_index("s") * SC_CORES + lax.axis_index("c")
    base = wid * ROWS_PER_W
    pltpu.sync_copy(idx_hbm.at[wid], idx_v)
    copies = []
    for j in range(CHUNKS_PER_W):
        copies.append(
            pltpu.async_copy(
                tables_hbm.at[idx_v.at[j]],
                rows_v.at[pl.ds(j * IDX_MINOR, IDX_MINOR)],
                sem,
            )
        )
    for c in copies:
        c.wait()
    pltpu.sync_copy(rows_v, out_hbm.at[pl.ds(base, ROWS_PER_W)])


@functools.cache
def _sc_gather_fn():
    return pl.kernel(
        _gather_body,
        out_type=jax.ShapeDtypeStruct((TOTAL_ROWS, ED), jnp.float32),
        mesh=plsc.VectorSubcoreMesh(
            core_axis_name="c", subcore_axis_name="s",
            num_cores=SC_CORES, num_subcores=SC_SUBCORES,
        ),
        scratch_types=[
            pltpu.VMEM((CHUNKS_PER_W, IDX_MINOR), jnp.int32),
            pltpu.VMEM((ROWS_PER_W, ED), jnp.float32),
            pltpu.SemaphoreType.DMA,
        ],
        compiler_params=pltpu.CompilerParams(use_tc_tiling_on_sc=False),
    )


def _mlp_body(x1_ref, xc_ref, w1a_ref, w1b_ref, b1_ref, w2_ref, b2_ref,
              w3_ref, b3_ref, g1_ref, be1_ref, g2_ref, be2_ref,
              g3_ref, be3_ref, out_ref):
    f32 = jnp.float32

    def bn(v, g, b):
        m = jnp.mean(v, axis=0, keepdims=True)
        var = jnp.mean((v - m) ** 2, axis=0, keepdims=True)
        return (v - m) * lax.rsqrt(var + EPS) * g + b

    xcn = bn(xc_ref[...], g1_ref[...], be1_ref[...])
    h = (jnp.dot(x1_ref[...], w1a_ref[...], preferred_element_type=f32)
         + jnp.dot(xcn, w1b_ref[...], preferred_element_type=f32)
         + b1_ref[...])
    h = jnp.maximum(h, 0.0)
    h = bn(h, g2_ref[...], be2_ref[...])
    h = jnp.dot(h, w2_ref[...], preferred_element_type=f32) + b2_ref[...]
    h = jnp.maximum(h, 0.0)
    h = bn(h, g3_ref[...], be3_ref[...])
    out_ref[...] = (jnp.dot(h, w3_ref[...], preferred_element_type=f32)
                    + b3_ref[...])


def _mlp(x1, xc, W1a, W1b, b1, W2, b2, W3, b3, g1, be1, g2, be2, g3, be3,
         interpret=False):
    return pl.pallas_call(
        _mlp_body,
        out_shape=jax.ShapeDtypeStruct((B, NC), jnp.float32),
        interpret=interpret,
    )(x1, xc, W1a, W1b, b1.reshape(1, L1), W2, b2.reshape(1, L2),
      W3, b3.reshape(1, NC), g1.reshape(1, NCONT), be1.reshape(1, NCONT),
      g2.reshape(1, L1), be2.reshape(1, L1), g3.reshape(1, L2),
      be3.reshape(1, L2))


def kernel(x, emb_tables, W1, b1, W2, b2, W3, b3, g1, be1, g2, be2, g3, be3):
    tables_flat = emb_tables.reshape(NCAT * VOCAB, ED)
    offsets = (jnp.arange(NCAT, dtype=jnp.int32) * VOCAB)[None, :]
    idx = x[:, :NCAT].astype(jnp.int32) + offsets
    idx3d = idx.reshape(NW, CHUNKS_PER_W, IDX_MINOR)
    gathered = _sc_gather_fn()(tables_flat, idx3d)
    x1 = gathered.reshape(B, N_EMB)
    xc = x[:, NCAT:]
    return _mlp(x1, xc, W1[:N_EMB], W1[N_EMB:], b1, W2, b2, W3, b3,
                g1, be1, g2, be2, g3, be3)
